# baseline (speedup 1.0000x reference)
import functools

import jax
import jax.numpy as jnp
from jax import lax
from jax.experimental import pallas as pl
from jax.experimental.pallas import tpu as pltpu
from jax.experimental.pallas import tpu_sc as plsc

N = 10000
E = 320000
D_IN = 128
D_HID = 64

NC = 2
NS = 16
NW = NC * NS
C = 128
NCH = 81
EPW = NCH * C
EPAD = NW * EPW
NP = 10240
RPT = NP // NS

_mesh = plsc.VectorSubcoreMesh(core_axis_name="c", subcore_axis_name="s",
                               num_cores=NC, num_subcores=NS)
_sc_params = pltpu.CompilerParams(use_tc_tiling_on_sc=False,
                                  needs_layout_passes=False)


def _worker_id():
    return lax.axis_index("s") * NC + lax.axis_index("c")


@functools.partial(
    pl.kernel,
    out_type=jax.ShapeDtypeStruct((NC, NP), jnp.float32),
    mesh=_mesh,
    compiler_params=_sc_params,
    scratch_types=[
        pltpu.VMEM((NCH, C), jnp.int32),
        pltpu.VMEM((C,), jnp.float32),
        pltpu.VMEM_SHARED((NP,), jnp.float32),
        pltpu.SemaphoreType.DMA,
    ],
)
def _sc_degree(edges_hbm, out_hbm, dst_v, ones_v, acc_sh, sem):
    cax = lax.axis_index("c")
    s = lax.axis_index("s")
    wid = _worker_id()
    pltpu.sync_copy(edges_hbm.at[1, wid], dst_v)
    for j in range(C // 16):
        ones_v[pl.ds(j * 16, 16)] = jnp.zeros((16,), jnp.float32)
    for k in range(RPT // C):
        pltpu.sync_copy(ones_v, acc_sh.at[pl.ds(s * RPT + k * C, C)])
    for j in range(C // 16):
        ones_v[pl.ds(j * 16, 16)] = jnp.ones((16,), jnp.float32)
    plsc.subcore_barrier()

    B = 3

    def body(k, carry):
        for i in range(B):
            pltpu.async_copy(ones_v, acc_sh.at[dst_v.at[k * B + i]], sem,
                             add=True)
        return carry

    def body_drain(k, carry):
        for i in range(B):
            pltpu.async_copy(ones_v, acc_sh.at[dst_v.at[k * B + i]], sem,
                             add=True)
        for i in range(B):
            pltpu.make_async_copy(ones_v, acc_sh.at[dst_v.at[0]], sem).wait()
        return carry

    body(0, 0)
    lax.fori_loop(1, NCH // B, body_drain, 0)
    for i in range(B):
        pltpu.make_async_copy(ones_v, acc_sh.at[dst_v.at[0]], sem).wait()
    plsc.subcore_barrier()
    pltpu.sync_copy(acc_sh.at[pl.ds(s * RPT, RPT)],
                    out_hbm.at[cax, pl.ds(s * RPT, RPT)])


@functools.partial(
    pl.kernel,
    out_type=jax.ShapeDtypeStruct((NC, NP, D_HID), jnp.float32),
    mesh=_mesh,
    compiler_params=_sc_params,
    scratch_types=[
        pltpu.VMEM((NCH, C), jnp.int32),
        pltpu.VMEM((NCH, C), jnp.int32),
        [pltpu.VMEM((C, D_HID), jnp.float32) for _ in range(4)],
        [pltpu.SemaphoreType.DMA for _ in range(4)],
        [pltpu.SemaphoreType.DMA for _ in range(4)],
        pltpu.VMEM_SHARED((NP, D_HID), jnp.float32),
    ],
)
def _sc_msg64(hp_hbm, edges_hbm, zeros2_hbm, out_hbm,
              src_v, dst_v, rows, gsem, ssem, acc_sh):
    cax = lax.axis_index("c")
    s = lax.axis_index("s")
    wid = _worker_id()
    pltpu.sync_copy(edges_hbm.at[0, wid], src_v)
    pltpu.sync_copy(edges_hbm.at[1, wid], dst_v)
    pltpu.sync_copy(zeros2_hbm.at[pl.ds(s * RPT, RPT)],
                    acc_sh.at[pl.ds(s * RPT, RPT)])
    plsc.subcore_barrier()

    def issue_g(c, b):
        pltpu.async_copy(hp_hbm.at[src_v.at[c]], rows[b], gsem[b])

    def wait_g(c, b):
        pltpu.make_async_copy(hp_hbm.at[src_v.at[c]], rows[b], gsem[b]).wait()

    def issue_s(c, b):
        pltpu.async_copy(rows[b], acc_sh.at[dst_v.at[c]], ssem[b], add=True)

    def wait_s(b):
        pltpu.make_async_copy(rows[b], acc_sh.at[dst_v.at[0]], ssem[b]).wait()

    issue_g(0, 0)
    issue_g(1, 1)
    issue_g(2, 2)
    wait_g(0, 0)
    issue_s(0, 0)
    issue_g(3, 3)
    wait_g(1, 1)
    issue_s(1, 1)

    def body(k, carry):
        c = 4 * k
        wait_s(0)
        issue_g(c + 4, 0)
        wait_g(c + 2, 2)
        issue_s(c + 2, 2)
        wait_s(1)
        issue_g(c + 5, 1)
        wait_g(c + 3, 3)
        issue_s(c + 3, 3)
        wait_s(2)
        issue_g(c + 6, 2)
        wait_g(c + 4, 0)
        issue_s(c + 4, 0)
        wait_s(3)
        issue_g(c + 7, 3)
        wait_g(c + 5, 1)
        issue_s(c + 5, 1)
        return carry

    lax.fori_loop(0, (NCH - 5) // 4, body, 0)
    wait_s(0)
    issue_g(NCH - 1, 0)
    wait_g(NCH - 3, 2)
    issue_s(NCH - 3, 2)
    wait_s(1)
    wait_g(NCH - 2, 3)
    issue_s(NCH - 2, 3)
    wait_s(2)
    wait_g(NCH - 1, 0)
    issue_s(NCH - 1, 0)
    wait_s(3)
    wait_s(0)
    plsc.subcore_barrier()
    pltpu.sync_copy(acc_sh.at[pl.ds(s * RPT, RPT)],
                    out_hbm.at[cax, pl.ds(s * RPT, RPT)])


@functools.partial(
    pl.kernel,
    out_type=jax.ShapeDtypeStruct((NC, NP), jnp.float32),
    mesh=_mesh,
    compiler_params=_sc_params,
    scratch_types=[
        pltpu.VMEM((NCH, C), jnp.int32),
        pltpu.VMEM((NCH, C), jnp.int32),
        pltpu.VMEM((N,), jnp.float32),
        [pltpu.VMEM((C,), jnp.float32) for _ in range(2)],
        [pltpu.SemaphoreType.DMA for _ in range(2)],
        pltpu.VMEM_SHARED((NP,), jnp.float32),
    ],
)
def _sc_msg1(gp_hbm, edges_hbm, out_hbm,
             src_v, dst_v, gtab_v, vals, sems, acc_sh):
    cax = lax.axis_index("c")
    s = lax.axis_index("s")
    wid = _worker_id()
    pltpu.sync_copy(edges_hbm.at[0, wid], src_v)
    pltpu.sync_copy(edges_hbm.at[1, wid], dst_v)
    pltpu.sync_copy(gp_hbm, gtab_v)
    for j in range(C // 16):
        vals[0][pl.ds(j * 16, 16)] = jnp.zeros((16,), jnp.float32)
    for k in range(RPT // C):
        pltpu.sync_copy(vals[0], acc_sh.at[pl.ds(s * RPT + k * C, C)])
    plsc.subcore_barrier()

    def compute(ch, b):
        for j in range(C // 16):
            idx = src_v[ch, pl.ds(j * 16, 16)]
            vals[b][pl.ds(j * 16, 16)] = plsc.load_gather(gtab_v, [idx])

    def issue_s(ch, b):
        pltpu.async_copy(vals[b], acc_sh.at[dst_v.at[ch]], sems[b], add=True)

    def wait_s(b):
        pltpu.make_async_copy(vals[b], acc_sh.at[dst_v.at[0]], sems[b]).wait()

    compute(0, 0)
    issue_s(0, 0)
    compute(1, 1)
    issue_s(1, 1)

    def body(k, carry):
        wait_s(0)
        compute(2 * k, 0)
        issue_s(2 * k, 0)
        wait_s(1)
        compute(2 * k + 1, 1)
        issue_s(2 * k + 1, 1)
        return carry

    lax.fori_loop(1, NCH // 2, body, 0)
    wait_s(0)
    compute(NCH - 1, 0)
    issue_s(NCH - 1, 0)
    wait_s(1)
    wait_s(0)
    plsc.subcore_barrier()
    pltpu.sync_copy(acc_sh.at[pl.ds(s * RPT, RPT)],
                    out_hbm.at[cax, pl.ds(s * RPT, RPT)])


def _tc1a_body(x_ref, w1_ref, h_ref):
    h_ref[...] = jnp.dot(x_ref[...], w1_ref[...],
                         preferred_element_type=jnp.float32)


def _tc1b_body(cntp_ref, h_ref, hp_ref, dis_ref):
    cnt = cntp_ref[0:1, :] + cntp_ref[1:2, :]
    dis_row = lax.rsqrt(cnt + 1.0)
    dis = dis_row.reshape(NP, 1)[0:N]
    hp_ref[...] = h_ref[...] * dis
    dis_ref[...] = dis_row[:, 0:N]


def _tc2_body(sp_ref, hp_ref, dis_ref, b1_ref, w2_ref, gp_ref):
    dis = dis_ref[...].reshape(N, 1)
    sacc = sp_ref[0, 0:N, :] + sp_ref[1, 0:N, :] + hp_ref[...]
    h1 = jnp.maximum(dis * sacc + b1_ref[...], 0.0)
    h2 = jnp.dot(h1, w2_ref[...], preferred_element_type=jnp.float32)
    gp_ref[...] = (dis * h2).reshape(1, N)


def _tc3_body(tp_ref, gp_ref, dis_ref, b2_ref, out_ref):
    t = tp_ref[0:1, 0:N] + tp_ref[1:2, 0:N] + gp_ref[...]
    out_ref[...] = jax.nn.sigmoid(dis_ref[...] * t + b2_ref[...])


_tc1a = pl.pallas_call(
    _tc1a_body,
    out_shape=jax.ShapeDtypeStruct((N, D_HID), jnp.float32),
)
_tc1b = pl.pallas_call(
    _tc1b_body,
    out_shape=[jax.ShapeDtypeStruct((N, D_HID), jnp.float32),
               jax.ShapeDtypeStruct((1, N), jnp.float32)],
)
_tc2 = pl.pallas_call(
    _tc2_body,
    out_shape=jax.ShapeDtypeStruct((1, N), jnp.float32),
)
_tc3 = pl.pallas_call(
    _tc3_body,
    out_shape=jax.ShapeDtypeStruct((1, N), jnp.float32),
)


def kernel(x, edge_index, W1, b1, W2, b2):
    npad = EPAD - E
    pad_src = (jnp.arange(npad, dtype=jnp.int32) * 97) % N
    pad_dst = N + (jnp.arange(npad, dtype=jnp.int32) % (NP - N))
    edges_p = jnp.concatenate(
        [edge_index, jnp.stack([pad_src, pad_dst])], axis=1)
    edges_r = edges_p.reshape(2, NW, NCH, C)
    zeros2 = jnp.zeros((NP, D_HID), jnp.float32)

    h = _tc1a(x, W1)
    cntp = _sc_degree(edges_r)
    hp, dis = _tc1b(cntp, h)
    sp = _sc_msg64(hp, edges_r, zeros2)
    gp = _tc2(sp, hp, dis, b1.reshape(1, D_HID), W2)
    tp = _sc_msg1(gp.reshape(N), edges_r)
    out = _tc3(tp, gp, dis, b2.reshape(1, 1))
    return out.reshape(N, 1)


# speedup vs baseline: 1.0298x; 1.0298x over previous
import functools

import jax
import jax.numpy as jnp
from jax import lax
from jax.experimental import pallas as pl
from jax.experimental.pallas import tpu as pltpu
from jax.experimental.pallas import tpu_sc as plsc

N = 10000
E = 320000
D_IN = 128
D_HID = 64

NC = 2
NS = 16
NW = NC * NS
EPW = E // NW
C = 80
NCH = EPW // C
NP = 10240
RPT = NP // NS

_mesh = plsc.VectorSubcoreMesh(core_axis_name="c", subcore_axis_name="s",
                               num_cores=NC, num_subcores=NS)
_sc_params = pltpu.CompilerParams(use_tc_tiling_on_sc=False,
                                  needs_layout_passes=False)


def _worker_id():
    return lax.axis_index("s") * NC + lax.axis_index("c")


@functools.partial(
    pl.kernel,
    out_type=jax.ShapeDtypeStruct((NC, NP), jnp.float32),
    mesh=_mesh,
    compiler_params=_sc_params,
    scratch_types=[
        pltpu.VMEM((NCH, C), jnp.int32),
        pltpu.VMEM((C,), jnp.float32),
        pltpu.VMEM_SHARED((NP,), jnp.float32),
        pltpu.SemaphoreType.DMA,
    ],
)
def _sc_degree(edges_hbm, out_hbm, dst_v, ones_v, acc_sh, sem):
    cax = lax.axis_index("c")
    s = lax.axis_index("s")
    wid = _worker_id()
    pltpu.sync_copy(edges_hbm.at[1, wid], dst_v)
    for j in range(C // 16):
        ones_v[pl.ds(j * 16, 16)] = jnp.zeros((16,), jnp.float32)
    for k in range(RPT // C):
        pltpu.sync_copy(ones_v, acc_sh.at[pl.ds(s * RPT + k * C, C)])
    for j in range(C // 16):
        ones_v[pl.ds(j * 16, 16)] = jnp.ones((16,), jnp.float32)
    plsc.subcore_barrier()

    B = 5

    def body(k, carry):
        for i in range(B):
            pltpu.async_copy(ones_v, acc_sh.at[dst_v.at[k * B + i]], sem,
                             add=True)
        return carry

    def body_drain(k, carry):
        for i in range(B):
            pltpu.async_copy(ones_v, acc_sh.at[dst_v.at[k * B + i]], sem,
                             add=True)
        for i in range(B):
            pltpu.make_async_copy(ones_v, acc_sh.at[dst_v.at[0]], sem).wait()
        return carry

    body(0, 0)
    lax.fori_loop(1, NCH // B, body_drain, 0)
    for i in range(B):
        pltpu.make_async_copy(ones_v, acc_sh.at[dst_v.at[0]], sem).wait()
    plsc.subcore_barrier()
    pltpu.sync_copy(acc_sh.at[pl.ds(s * RPT, RPT)],
                    out_hbm.at[cax, pl.ds(s * RPT, RPT)])


@functools.partial(
    pl.kernel,
    out_type=jax.ShapeDtypeStruct((NC, NP, D_HID), jnp.float32),
    mesh=_mesh,
    compiler_params=_sc_params,
    scratch_types=[
        pltpu.VMEM((NCH, C), jnp.int32),
        pltpu.VMEM((NCH, C), jnp.int32),
        [pltpu.VMEM((C, D_HID), jnp.float32) for _ in range(4)],
        [pltpu.SemaphoreType.DMA for _ in range(4)],
        [pltpu.SemaphoreType.DMA for _ in range(4)],
        pltpu.VMEM_SHARED((NP, D_HID), jnp.float32),
    ],
)
def _sc_msg64(hp_hbm, edges_hbm, zeros2_hbm, out_hbm,
              src_v, dst_v, rows, gsem, ssem, acc_sh):
    cax = lax.axis_index("c")
    s = lax.axis_index("s")
    wid = _worker_id()
    cp0 = pltpu.async_copy(edges_hbm.at[0, wid], src_v, gsem[0])
    cp1 = pltpu.async_copy(edges_hbm.at[1, wid], dst_v, gsem[1])
    cp2 = pltpu.async_copy(zeros2_hbm.at[pl.ds(s * RPT, RPT)],
                           acc_sh.at[pl.ds(s * RPT, RPT)], gsem[2])
    cp0.wait()
    cp1.wait()
    cp2.wait()
    plsc.subcore_barrier()

    def issue_g(c, b):
        pltpu.async_copy(hp_hbm.at[src_v.at[c]], rows[b], gsem[b])

    def wait_g(c, b):
        pltpu.make_async_copy(hp_hbm.at[src_v.at[c]], rows[b], gsem[b]).wait()

    def issue_s(c, b):
        pltpu.async_copy(rows[b], acc_sh.at[dst_v.at[c]], ssem[b], add=True)

    def wait_s(b):
        pltpu.make_async_copy(rows[b], acc_sh.at[dst_v.at[0]], ssem[b]).wait()

    issue_g(0, 0)
    issue_g(1, 1)
    issue_g(2, 2)
    wait_g(0, 0)
    issue_s(0, 0)
    issue_g(3, 3)
    wait_g(1, 1)
    issue_s(1, 1)

    def body(k, carry):
        c = 4 * k
        wait_s(0)
        issue_g(c + 4, 0)
        wait_g(c + 2, 2)
        issue_s(c + 2, 2)
        wait_s(1)
        issue_g(c + 5, 1)
        wait_g(c + 3, 3)
        issue_s(c + 3, 3)
        wait_s(2)
        issue_g(c + 6, 2)
        wait_g(c + 4, 0)
        issue_s(c + 4, 0)
        wait_s(3)
        issue_g(c + 7, 3)
        wait_g(c + 5, 1)
        issue_s(c + 5, 1)
        return carry

    lax.fori_loop(0, (NCH - 5) // 4, body, 0)
    wait_s(0)
    issue_g(124, 0)
    wait_g(122, 2)
    issue_s(122, 2)
    wait_s(1)
    wait_g(123, 3)
    issue_s(123, 3)
    wait_s(2)
    wait_g(124, 0)
    issue_s(124, 0)
    wait_s(3)
    wait_s(0)
    plsc.subcore_barrier()
    pltpu.sync_copy(acc_sh.at[pl.ds(s * RPT, RPT)],
                    out_hbm.at[cax, pl.ds(s * RPT, RPT)])


@functools.partial(
    pl.kernel,
    out_type=jax.ShapeDtypeStruct((NC, NP), jnp.float32),
    mesh=_mesh,
    compiler_params=_sc_params,
    scratch_types=[
        pltpu.VMEM((NCH, C), jnp.int32),
        pltpu.VMEM((NCH, C), jnp.int32),
        pltpu.VMEM((N,), jnp.float32),
        [pltpu.VMEM((C,), jnp.float32) for _ in range(2)],
        [pltpu.SemaphoreType.DMA for _ in range(2)],
        pltpu.VMEM_SHARED((NP,), jnp.float32),
    ],
)
def _sc_msg1(gp_hbm, edges_hbm, out_hbm,
             src_v, dst_v, gtab_v, vals, sems, acc_sh):
    cax = lax.axis_index("c")
    s = lax.axis_index("s")
    wid = _worker_id()
    cp0 = pltpu.async_copy(edges_hbm.at[0, wid], src_v, sems[0])
    cp1 = pltpu.async_copy(edges_hbm.at[1, wid], dst_v, sems[1])
    pltpu.sync_copy(gp_hbm, gtab_v)
    cp0.wait()
    cp1.wait()
    for j in range(C // 16):
        vals[0][pl.ds(j * 16, 16)] = jnp.zeros((16,), jnp.float32)
    for k in range(RPT // C):
        pltpu.sync_copy(vals[0], acc_sh.at[pl.ds(s * RPT + k * C, C)])
    plsc.subcore_barrier()

    def compute(ch, b):
        for j in range(C // 16):
            idx = src_v[ch, pl.ds(j * 16, 16)]
            vals[b][pl.ds(j * 16, 16)] = plsc.load_gather(gtab_v, [idx])

    def issue_s(ch, b):
        pltpu.async_copy(vals[b], acc_sh.at[dst_v.at[ch]], sems[b], add=True)

    def wait_s(b):
        pltpu.make_async_copy(vals[b], acc_sh.at[dst_v.at[0]], sems[b]).wait()

    compute(0, 0)
    issue_s(0, 0)
    compute(1, 1)
    issue_s(1, 1)

    def body(k, carry):
        wait_s(0)
        compute(2 * k, 0)
        issue_s(2 * k, 0)
        wait_s(1)
        compute(2 * k + 1, 1)
        issue_s(2 * k + 1, 1)
        return carry

    lax.fori_loop(1, NCH // 2, body, 0)
    wait_s(0)
    compute(NCH - 1, 0)
    issue_s(NCH - 1, 0)
    wait_s(1)
    wait_s(0)
    plsc.subcore_barrier()
    pltpu.sync_copy(acc_sh.at[pl.ds(s * RPT, RPT)],
                    out_hbm.at[cax, pl.ds(s * RPT, RPT)])


def _tc1a_body(x_ref, w1_ref, h_ref):
    h_ref[...] = jnp.dot(x_ref[...], w1_ref[...],
                         preferred_element_type=jnp.float32)


def _tc1b_body(cntp_ref, h_ref, hp_ref, dis_ref):
    cnt = cntp_ref[0:1, :] + cntp_ref[1:2, :]
    dis_row = lax.rsqrt(cnt + 1.0)
    dis = dis_row.reshape(NP, 1)[0:N]
    hp_ref[...] = h_ref[...] * dis
    dis_ref[...] = dis_row[:, 0:N]


def _tc2_body(sp_ref, hp_ref, dis_ref, b1_ref, w2_ref, gp_ref):
    dis = dis_ref[...].reshape(N, 1)
    sacc = sp_ref[0, 0:N, :] + sp_ref[1, 0:N, :] + hp_ref[...]
    h1 = jnp.maximum(dis * sacc + b1_ref[...], 0.0)
    h2 = jnp.dot(h1, w2_ref[...], preferred_element_type=jnp.float32)
    gp_ref[...] = (dis * h2).reshape(1, N)


def _tc3_body(tp_ref, gp_ref, dis_ref, b2_ref, out_ref):
    t = tp_ref[0:1, 0:N] + tp_ref[1:2, 0:N] + gp_ref[...]
    out_ref[...] = jax.nn.sigmoid(dis_ref[...] * t + b2_ref[...])


_tc1a = pl.pallas_call(
    _tc1a_body,
    out_shape=jax.ShapeDtypeStruct((N, D_HID), jnp.float32),
)
_tc1b = pl.pallas_call(
    _tc1b_body,
    out_shape=[jax.ShapeDtypeStruct((N, D_HID), jnp.float32),
               jax.ShapeDtypeStruct((1, N), jnp.float32)],
)
_tc2 = pl.pallas_call(
    _tc2_body,
    out_shape=jax.ShapeDtypeStruct((1, N), jnp.float32),
)
_tc3 = pl.pallas_call(
    _tc3_body,
    out_shape=jax.ShapeDtypeStruct((1, N), jnp.float32),
)


def kernel(x, edge_index, W1, b1, W2, b2):
    edges_r = edge_index.reshape(2, NW, NCH, C)
    zeros2 = jnp.zeros((NP, D_HID), jnp.float32)

    h = _tc1a(x, W1)
    cntp = _sc_degree(edges_r)
    hp, dis = _tc1b(cntp, h)
    sp = _sc_msg64(hp, edges_r, zeros2)
    gp = _tc2(sp, hp, dis, b1.reshape(1, D_HID), W2)
    tp = _sc_msg1(gp.reshape(N), edges_r)
    out = _tc3(tp, gp, dis, b2.reshape(1, 1))
    return out.reshape(N, 1)
